# baseline (speedup 1.0000x reference)
import functools
import jax
import jax.numpy as jnp
from jax.experimental import pallas as pl

N = 10000
E = 320000
G = 64
HW = 128
DEPTH = 4
RATIO = 0.5

NP = 10240
BN_ROWS = 1024
GRID = NP // BN_ROWS


def _relu_mm_body(x_ref, w_ref, b_ref, o_ref):
    h = jnp.maximum(x_ref[...], 0.0)
    o_ref[...] = jnp.dot(h, w_ref[...], preferred_element_type=jnp.float32) + b_ref[...]


def _relu_mm(x_pad, W, b):
    return pl.pallas_call(
        _relu_mm_body,
        grid=(GRID,),
        in_specs=[
            pl.BlockSpec((BN_ROWS, HW), lambda i: (i, 0)),
            pl.BlockSpec((HW, HW), lambda i: (0, 0)),
            pl.BlockSpec((1, HW), lambda i: (0, 0)),
        ],
        out_specs=pl.BlockSpec((BN_ROWS, HW), lambda i: (i, 0)),
        out_shape=jax.ShapeDtypeStruct((NP, HW), jnp.float32),
    )(x_pad, W, b.reshape(1, HW))


def _mm_body(x_ref, w_ref, o_ref):
    o_ref[...] = jnp.dot(x_ref[...], w_ref[...], preferred_element_type=jnp.float32)


def _mm(x_pad, W):
    return pl.pallas_call(
        _mm_body,
        grid=(GRID,),
        in_specs=[
            pl.BlockSpec((BN_ROWS, HW), lambda i: (i, 0)),
            pl.BlockSpec((HW, HW), lambda i: (0, 0)),
        ],
        out_specs=pl.BlockSpec((BN_ROWS, HW), lambda i: (i, 0)),
        out_shape=jax.ShapeDtypeStruct((NP, HW), jnp.float32),
    )(x_pad, W)


def _gap_body(moh_ref, x_ref, o_ref):
    @pl.when(pl.program_id(0) == 0)
    def _():
        o_ref[...] = jnp.zeros_like(o_ref)
    o_ref[...] += jnp.dot(moh_ref[...].T, x_ref[...],
                          preferred_element_type=jnp.float32)


def _gap_sums(moh_pad, x_pad):
    return pl.pallas_call(
        _gap_body,
        grid=(GRID,),
        in_specs=[
            pl.BlockSpec((BN_ROWS, G), lambda i: (i, 0)),
            pl.BlockSpec((BN_ROWS, HW), lambda i: (i, 0)),
        ],
        out_specs=pl.BlockSpec((G, HW), lambda i: (0, 0)),
        out_shape=jax.ShapeDtypeStruct((G, HW), jnp.float32),
    )(moh_pad, x_pad)


def _head_body(r_ref, w1_ref, b1_ref, g_ref, be_ref, w2_ref, b2_ref, o_ref):
    h = jnp.dot(r_ref[...], w1_ref[...], preferred_element_type=jnp.float32) + b1_ref[...]
    mean = jnp.mean(h, axis=0, keepdims=True)
    var = jnp.mean((h - mean) ** 2, axis=0, keepdims=True)
    h = (h - mean) / jnp.sqrt(var + 1e-5) * g_ref[...] + be_ref[...]
    h = jnp.maximum(h, 0.0)
    o_ref[...] = jnp.dot(h, w2_ref[...], preferred_element_type=jnp.float32) + b2_ref[...]


def _head(r, w1, b1, g, be, w2p, b2p):
    hw2 = HW // 2
    return pl.pallas_call(
        _head_body,
        in_specs=[
            pl.BlockSpec((G, HW), lambda: (0, 0)),
            pl.BlockSpec((HW, hw2), lambda: (0, 0)),
            pl.BlockSpec((1, hw2), lambda: (0, 0)),
            pl.BlockSpec((1, hw2), lambda: (0, 0)),
            pl.BlockSpec((1, hw2), lambda: (0, 0)),
            pl.BlockSpec((hw2, HW), lambda: (0, 0)),
            pl.BlockSpec((1, HW), lambda: (0, 0)),
        ],
        out_specs=pl.BlockSpec((G, HW), lambda: (0, 0)),
        out_shape=jax.ShapeDtypeStruct((G, HW), jnp.float32),
    )(r, w1, b1.reshape(1, hw2), g.reshape(1, hw2), be.reshape(1, hw2),
      w2p, b2p)


def _pad_rows(a):
    return jnp.pad(a, ((0, NP - N), (0, 0)))


def kernel(x, edge_index, batch, t_gamma, t_beta, t_W, t_b, c_W, c_b, p_w,
           pr_W1, pr_b1, pr_g, pr_be, pr_W2, pr_b2):
    src = edge_index[0]
    dst = edge_index[1]
    node_mask = jnp.ones((N,), bool)
    edge_mask = jnp.ones((E,), bool)

    x_pad = _pad_rows(x)

    for i in range(DEPTH):
        for j in (2 * i, 2 * i + 1):
            m = node_mask.astype(jnp.float32)[:, None]
            cnt = jnp.maximum(jnp.sum(m), 1.0)
            xs = x_pad[:N]
            mean = jnp.sum(xs * m, axis=0) / cnt
            var = jnp.sum(((xs - mean) ** 2) * m, axis=0) / cnt
            hbn = (xs - mean) / jnp.sqrt(var + 1e-5) * t_gamma[j] + t_beta[j]
            h_pad = _relu_mm(_pad_rows(hbn), t_W[j], t_b[j])

            hc_pad = _mm(h_pad, c_W[j])
            h = hc_pad[:N]
            ew = edge_mask.astype(jnp.float32)
            nm = node_mask.astype(jnp.float32)
            deg = jax.ops.segment_sum(ew, dst, num_segments=N) + 2.0 * nm
            dis = jnp.where(deg > 0, 1.0 / jnp.sqrt(jnp.maximum(deg, 1e-12)), 0.0)
            norm = dis[src] * ew * dis[dst]
            out = jax.ops.segment_sum(h[src] * norm[:, None], dst, num_segments=N)
            out = out + h * (2.0 * dis * dis * nm)[:, None] + c_b[j]
            x_pad = _pad_rows(out)

        xs = x_pad[:N]
        w = p_w[i]
        score = jnp.tanh((xs @ w) / jnp.maximum(jnp.linalg.norm(w), 1e-12))
        skey = jnp.where(node_mask, score, -jnp.inf)
        order = jnp.lexsort((-skey, batch))
        bs = batch[order]
        counts = jax.ops.segment_sum(jnp.ones((N,), jnp.int32), batch, num_segments=G)
        starts = jnp.cumsum(counts) - counts
        pos = jnp.arange(N) - starts[bs]
        n_act = jax.ops.segment_sum(node_mask.astype(jnp.int32), batch, num_segments=G)
        k = jnp.ceil(RATIO * n_act.astype(jnp.float32)).astype(jnp.int32)
        keep = (pos < k[bs]) & node_mask[order]
        node_mask = jnp.zeros((N,), bool).at[order].set(keep)
        xs = jnp.where(node_mask[:, None], xs * score[:, None], 0.0)
        edge_mask = edge_mask & node_mask[src] & node_mask[dst]
        x_pad = _pad_rows(xs)

    oh = (batch[:, None] == jnp.arange(G)[None, :]).astype(jnp.float32)
    moh = oh * node_mask.astype(jnp.float32)[:, None]
    moh_pad = jnp.pad(moh, ((0, NP - N), (0, 0)))
    sums = _gap_sums(moh_pad, x_pad)
    cnts = jnp.maximum(jnp.sum(moh, axis=0), 1.0)
    r = sums / cnts[:, None]

    w2p = jnp.pad(pr_W2, ((0, 0), (0, HW - pr_W2.shape[1])))
    b2p = jnp.pad(pr_b2, (0, HW - pr_b2.shape[0])).reshape(1, HW)
    out = _head(r, pr_W1, pr_b1, pr_g, pr_be, w2p, b2p)
    return out[:, :pr_W2.shape[1]]
